# baseline (speedup 1.0000x reference)
import functools
import numpy as np
import jax
import jax.numpy as jnp
from jax import lax
from jax.experimental import pallas as pl
from jax.experimental.pallas import tpu as pltpu
from jax.experimental.pallas import tpu_sc as plsc

_DIM = 16
_K = 8
_TB = 3.0
_MIN_BW = 1e-3
_MIN_BH = 1e-3
_MIN_D = 1e-3
_PAD_C = float(np.log(np.exp(1 - _MIN_D) - 1))

_OFF_EW = 0
_OFF_RW = 144
_OFF_H = 272
_OFF_EH = 400
_OFF_D = 528
_TAB = 672

_NW = 32
_CH = 512

_LOGC = (3.5075520532e-06, 0.99979243573, -0.49697791117, 0.31459053537,
         -0.18878267362, 0.081726808373, -0.017208061121)
_LN2 = 0.6931471805599453


def _tab_body(uwT_ref, uhT_ref, udT_ref, tab_ref):
    def edges(uT, min_b):
        m = jnp.max(uT, axis=0, keepdims=True)
        e = jnp.exp(uT - m)
        w = min_b + (1 - min_b * _K) * (e / jnp.sum(e, axis=0, keepdims=True))
        ed = [jnp.full((1, _DIM), -_TB, jnp.float32)]
        acc = jnp.zeros((1, _DIM), jnp.float32)
        for k in range(_K - 1):
            acc = acc + w[k : k + 1, :]
            ed.append(2 * _TB * acc - _TB)
        ed.append(jnp.full((1, _DIM), _TB, jnp.float32))
        return ed

    ew = edges(uwT_ref[...], _MIN_BW)
    eh = edges(uhT_ref[...], _MIN_BH)
    ud = udT_ref[...]
    pad = jnp.full((1, _DIM), _PAD_C, jnp.float32)
    ud_rows = [pad] + [ud[k : k + 1, :] for k in range(_K - 1)] + [pad]
    dv = [_MIN_D + jnp.log1p(jnp.exp(u)) for u in ud_rows]

    def put(off, row):
        tab_ref[pl.ds(off, _DIM)] = row.reshape(_DIM)

    for k in range(_K + 1):
        put(_OFF_EW + 16 * k, ew[k])
        put(_OFF_D + 16 * k, dv[k])
    for k in range(_K):
        put(_OFF_RW + 16 * k, 1.0 / (ew[k + 1] - ew[k]))
        put(_OFF_H + 16 * k, eh[k + 1] - eh[k])
        put(_OFF_EH + 16 * k, eh[k])


def _log16(r):
    bits = lax.bitcast_convert_type(r, jnp.int32)
    e = ((bits >> 23) & 0xFF) - 127
    m = lax.bitcast_convert_type(
        (bits & 0x007FFFFF) | 0x3F800000, jnp.float32)
    t = m - 1.0
    p = jnp.full((16,), _LOGC[6], jnp.float32)
    for c in (_LOGC[5], _LOGC[4], _LOGC[3], _LOGC[2], _LOGC[1], _LOGC[0]):
        p = p * t + c
    return e.astype(jnp.float32) * _LN2 + p


def _sc_body(tab_hbm, x_hbm, out_hbm, lad_hbm, tab, xb, ob, lb):
    cid = lax.axis_index("c")
    sid = lax.axis_index("s")
    wid = sid * 2 + cid
    pltpu.sync_copy(tab_hbm, tab)
    lane = lax.iota(jnp.int32, 16)
    ew = [tab[pl.ds(_OFF_EW + 16 * k, 16)] for k in range(1, _K)]

    rows_w = x_hbm.shape[0] // _NW
    nch = rows_w // _CH

    def chunk(c, carry):
        base = wid * rows_w + c * _CH
        pltpu.sync_copy(x_hbm.at[pl.ds(base, _CH)], xb)

        def row(i, carry2):
            x = xb[i]
            x_in = jnp.minimum(jnp.maximum(x, -_TB), _TB)
            bf = jnp.zeros((16,), jnp.float32)
            for e in ew:
                bf = bf + jnp.where(x_in >= e, 1.0, 0.0)
            idx = bf.astype(jnp.int32) * 16 + lane
            g_cumw = plsc.load_gather(tab, [idx + _OFF_EW])
            g_rw = plsc.load_gather(tab, [idx + _OFF_RW])
            g_h = plsc.load_gather(tab, [idx + _OFF_H])
            g_cumh = plsc.load_gather(tab, [idx + _OFF_EH])
            g_d = plsc.load_gather(tab, [idx + _OFF_D])
            g_d1 = plsc.load_gather(tab, [idx + _OFF_D + 16])

            g_delta = g_h * g_rw
            theta = (x_in - g_cumw) * g_rw
            omt = 1.0 - theta
            tomt = theta * omt
            th2 = theta * theta
            num = g_h * (g_delta * th2 + g_d * tomt)
            den = g_delta + (g_d + g_d1 - 2.0 * g_delta) * tomt
            rden = 1.0 / den
            out_in = g_cumh + num * rden
            dnum = (g_delta * g_delta) * (
                g_d1 * th2 + 2.0 * g_delta * tomt + g_d * (omt * omt))
            lad_in = _log16(dnum * rden * rden)

            inside = (x >= -_TB) & (x <= _TB)
            ob[i] = jnp.where(inside, out_in, x)
            lb[i] = jnp.where(inside, lad_in, 0.0)
            return carry2

        lax.fori_loop(0, _CH, row, 0)
        pltpu.sync_copy(ob, out_hbm.at[pl.ds(base, _CH)])
        pltpu.sync_copy(lb, lad_hbm.at[pl.ds(base, _CH)])
        return carry

    lax.fori_loop(0, nch, chunk, 0)


def kernel(x, unnormalized_widths, unnormalized_heights, unnormalized_derivatives):
    n, d = x.shape
    tab = pl.pallas_call(
        _tab_body,
        out_shape=jax.ShapeDtypeStruct((_TAB,), jnp.float32),
    )(unnormalized_widths.T, unnormalized_heights.T,
      unnormalized_derivatives.T)

    mesh = plsc.VectorSubcoreMesh(core_axis_name="c", subcore_axis_name="s")
    sck = pl.kernel(
        _sc_body,
        mesh=mesh,
        out_type=[
            jax.ShapeDtypeStruct((n, d), jnp.float32),
            jax.ShapeDtypeStruct((n, d), jnp.float32),
        ],
        scratch_types=[
            pltpu.VMEM((_TAB,), jnp.float32),
            pltpu.VMEM((_CH, _DIM), jnp.float32),
            pltpu.VMEM((_CH, _DIM), jnp.float32),
            pltpu.VMEM((_CH, _DIM), jnp.float32),
        ],
        compiler_params=pltpu.CompilerParams(
            needs_layout_passes=False, use_tc_tiling_on_sc=False),
    )
    out, lad = sck(tab, x)
    return out, lad


# speedup vs baseline: 1.2799x; 1.2799x over previous
import functools
import numpy as np
import jax
import jax.numpy as jnp
from jax import lax
from jax.experimental import pallas as pl
from jax.experimental.pallas import tpu as pltpu
from jax.experimental.pallas import tpu_sc as plsc

_DIM = 16
_K = 8
_TB = 3.0
_MIN_BW = 1e-3
_MIN_BH = 1e-3
_MIN_D = 1e-3
_PAD_C = float(np.log(np.exp(1 - _MIN_D) - 1))

_OFF_EW = 0
_OFF_RW = 144
_OFF_H = 272
_OFF_EH = 400
_OFF_D = 528
_TAB = 672

_NW = 32
_CH = 256

_LOGC = (3.5075520532e-06, 0.99979243573, -0.49697791117, 0.31459053537,
         -0.18878267362, 0.081726808373, -0.017208061121)
_LN2 = 0.6931471805599453


def _tab_body(uwT_ref, uhT_ref, udT_ref, tab_ref):
    def edges(uT, min_b):
        m = jnp.max(uT, axis=0, keepdims=True)
        e = jnp.exp(uT - m)
        w = min_b + (1 - min_b * _K) * (e / jnp.sum(e, axis=0, keepdims=True))
        ed = [jnp.full((1, _DIM), -_TB, jnp.float32)]
        acc = jnp.zeros((1, _DIM), jnp.float32)
        for k in range(_K - 1):
            acc = acc + w[k : k + 1, :]
            ed.append(2 * _TB * acc - _TB)
        ed.append(jnp.full((1, _DIM), _TB, jnp.float32))
        return ed

    ew = edges(uwT_ref[...], _MIN_BW)
    eh = edges(uhT_ref[...], _MIN_BH)
    ud = udT_ref[...]
    pad = jnp.full((1, _DIM), _PAD_C, jnp.float32)
    ud_rows = [pad] + [ud[k : k + 1, :] for k in range(_K - 1)] + [pad]
    dv = [_MIN_D + jnp.log1p(jnp.exp(u)) for u in ud_rows]

    def put(off, row):
        tab_ref[pl.ds(off, _DIM)] = row.reshape(_DIM)

    for k in range(_K + 1):
        put(_OFF_EW + 16 * k, ew[k])
        put(_OFF_D + 16 * k, dv[k])
    for k in range(_K):
        put(_OFF_RW + 16 * k, 1.0 / (ew[k + 1] - ew[k]))
        put(_OFF_H + 16 * k, eh[k + 1] - eh[k])
        put(_OFF_EH + 16 * k, eh[k])


def _log16(r):
    bits = lax.bitcast_convert_type(r, jnp.int32)
    e = ((bits >> 23) & 0xFF) - 127
    m = lax.bitcast_convert_type(
        (bits & 0x007FFFFF) | 0x3F800000, jnp.float32)
    t = m - 1.0
    p = jnp.full((16,), _LOGC[6], jnp.float32)
    for c in (_LOGC[5], _LOGC[4], _LOGC[3], _LOGC[2], _LOGC[1], _LOGC[0]):
        p = p * t + c
    return e.astype(jnp.float32) * _LN2 + p


def _sc_body(tab_hbm, x_hbm, out_hbm, lad_hbm, tab, xb, ob, lb, sem):
    cid = lax.axis_index("c")
    sid = lax.axis_index("s")
    wid = sid * 2 + cid
    pltpu.sync_copy(tab_hbm, tab)
    lane = lax.iota(jnp.int32, 16)
    ew = [tab[pl.ds(_OFF_EW + 16 * k, 16)] for k in range(1, _K)]

    rows_w = x_hbm.shape[0] // _NW
    nch = rows_w // _CH
    row_base = wid * rows_w

    def chunk(c, carry):
        base = row_base + c * _CH
        pltpu.async_copy(x_hbm.at[pl.ds(base, _CH)], xb, sem).wait()

        @plsc.parallel_loop(0, _CH, unroll=4)
        def row(i):
            x = xb[i]
            x_in = jnp.minimum(jnp.maximum(x, -_TB), _TB)
            bf = jnp.zeros((16,), jnp.float32)
            for e in ew:
                bf = bf + jnp.where(x_in >= e, 1.0, 0.0)
            idx = bf.astype(jnp.int32) * 16 + lane
            g_cumw = plsc.load_gather(tab, [idx + _OFF_EW])
            g_rw = plsc.load_gather(tab, [idx + _OFF_RW])
            g_h = plsc.load_gather(tab, [idx + _OFF_H])
            g_cumh = plsc.load_gather(tab, [idx + _OFF_EH])
            g_d = plsc.load_gather(tab, [idx + _OFF_D])
            g_d1 = plsc.load_gather(tab, [idx + _OFF_D + 16])

            g_delta = g_h * g_rw
            theta = (x_in - g_cumw) * g_rw
            omt = 1.0 - theta
            tomt = theta * omt
            th2 = theta * theta
            num = g_h * (g_delta * th2 + g_d * tomt)
            den = g_delta + (g_d + g_d1 - 2.0 * g_delta) * tomt
            rden = 1.0 / den
            out_in = g_cumh + num * rden
            dnum = (g_delta * g_delta) * (
                g_d1 * th2 + 2.0 * g_delta * tomt + g_d * (omt * omt))
            lad_in = _log16(dnum * rden * rden)

            inside = (x >= -_TB) & (x <= _TB)
            ob[i] = jnp.where(inside, out_in, x)
            lb[i] = jnp.where(inside, lad_in, 0.0)

        pltpu.async_copy(ob, out_hbm.at[pl.ds(base, _CH)], sem).wait()
        pltpu.async_copy(lb, lad_hbm.at[pl.ds(base, _CH)], sem).wait()
        return carry

    lax.fori_loop(0, nch, chunk, 0)


def kernel(x, unnormalized_widths, unnormalized_heights, unnormalized_derivatives):
    n, d = x.shape
    tab = pl.pallas_call(
        _tab_body,
        out_shape=jax.ShapeDtypeStruct((_TAB,), jnp.float32),
    )(unnormalized_widths.T, unnormalized_heights.T,
      unnormalized_derivatives.T)

    mesh = plsc.VectorSubcoreMesh(core_axis_name="c", subcore_axis_name="s")
    sck = pl.kernel(
        _sc_body,
        mesh=mesh,
        out_type=[
            jax.ShapeDtypeStruct((n, d), jnp.float32),
            jax.ShapeDtypeStruct((n, d), jnp.float32),
        ],
        scratch_types=[
            pltpu.VMEM((_TAB,), jnp.float32),
            pltpu.VMEM((_CH, _DIM), jnp.float32),
            pltpu.VMEM((_CH, _DIM), jnp.float32),
            pltpu.VMEM((_CH, _DIM), jnp.float32),
            pltpu.SemaphoreType.DMA,
        ],
        compiler_params=pltpu.CompilerParams(needs_layout_passes=False),
    )
    out, lad = sck(tab, x)
    return out, lad


# speedup vs baseline: 3.8426x; 3.0023x over previous
import numpy as np
import jax
import jax.numpy as jnp
from jax.experimental import pallas as pl
from jax.experimental.pallas import tpu as pltpu

_DIM = 16
_K = 8
_TB = 3.0
_MIN_BW = 1e-3
_MIN_BH = 1e-3
_MIN_D = 1e-3
_PAD_C = float(np.log(np.exp(1 - _MIN_D) - 1))

_BL = 16384


def _edges_from(u, min_b):
    m = jnp.max(u, axis=1, keepdims=True)
    e = jnp.exp(u - m)
    w = min_b + (1 - min_b * _K) * (e / jnp.sum(e, axis=1, keepdims=True))
    edges = [jnp.full((_DIM, 1), -_TB, dtype=u.dtype)]
    acc = jnp.zeros((_DIM, 1), dtype=u.dtype)
    for k in range(_K - 1):
        acc = acc + w[:, k : k + 1]
        edges.append(2 * _TB * acc - _TB)
    edges.append(jnp.full((_DIM, 1), _TB, dtype=u.dtype))
    return edges


def _body(uw_ref, uh_ref, ud_ref, x_ref, out_ref, lad_ref):
    f32 = jnp.float32
    ew = _edges_from(uw_ref[...], _MIN_BW)
    eh = _edges_from(uh_ref[...], _MIN_BH)
    widths = [ew[k + 1] - ew[k] for k in range(_K)]
    heights = [eh[k + 1] - eh[k] for k in range(_K)]
    rw = [1.0 / widths[k] for k in range(_K)]

    ud = ud_ref[...]
    pad = jnp.full((_DIM, 1), _PAD_C, dtype=f32)
    ud_cols = [pad] + [ud[:, k : k + 1] for k in range(_K - 1)] + [pad]
    derivs = [_MIN_D + jnp.log1p(jnp.exp(u)) for u in ud_cols]

    x = x_ref[...]
    inside = (x >= -_TB) & (x <= _TB)
    x_in = jnp.clip(x, -_TB, _TB)

    g_cumw = jnp.broadcast_to(ew[0], x.shape)
    g_rw = jnp.broadcast_to(rw[0], x.shape)
    g_h = jnp.broadcast_to(heights[0], x.shape)
    g_cumh = jnp.broadcast_to(eh[0], x.shape)
    g_d = jnp.broadcast_to(derivs[0], x.shape)
    g_d1 = jnp.broadcast_to(derivs[1], x.shape)
    for j in range(1, _K):
        m = (x_in >= ew[j]).astype(f32)
        g_cumw = g_cumw + (ew[j] - ew[j - 1]) * m
        g_rw = g_rw + (rw[j] - rw[j - 1]) * m
        g_h = g_h + (heights[j] - heights[j - 1]) * m
        g_cumh = g_cumh + (eh[j] - eh[j - 1]) * m
        g_d = g_d + (derivs[j] - derivs[j - 1]) * m
        g_d1 = g_d1 + (derivs[j + 1] - derivs[j]) * m

    g_delta = g_h * g_rw
    theta = (x_in - g_cumw) * g_rw
    omt = 1.0 - theta
    tomt = theta * omt
    th2 = theta * theta
    num = g_h * (g_delta * th2 + g_d * tomt)
    den = g_delta + (g_d + g_d1 - 2.0 * g_delta) * tomt
    rden = 1.0 / den
    out_in = g_cumh + num * rden
    dnum = (g_delta * g_delta) * (g_d1 * th2 + 2.0 * g_delta * tomt + g_d * (omt * omt))
    lad_in = jnp.log(dnum * rden * rden)

    out_ref[...] = jnp.where(inside, out_in, x)
    lad_ref[...] = jnp.where(inside, lad_in, 0.0)


def kernel(x, unnormalized_widths, unnormalized_heights, unnormalized_derivatives):
    n, d = x.shape
    xt = x.T

    grid = (n // _BL,)
    out_t, lad_t = pl.pallas_call(
        _body,
        grid=grid,
        in_specs=[
            pl.BlockSpec((_DIM, _K), lambda i: (0, 0)),
            pl.BlockSpec((_DIM, _K), lambda i: (0, 0)),
            pl.BlockSpec((_DIM, _K - 1), lambda i: (0, 0)),
            pl.BlockSpec((_DIM, _BL), lambda i: (0, i)),
        ],
        out_specs=[
            pl.BlockSpec((_DIM, _BL), lambda i: (0, i)),
            pl.BlockSpec((_DIM, _BL), lambda i: (0, i)),
        ],
        out_shape=[
            jax.ShapeDtypeStruct((d, n), jnp.float32),
            jax.ShapeDtypeStruct((d, n), jnp.float32),
        ],
        compiler_params=pltpu.CompilerParams(
            dimension_semantics=("arbitrary",),
        ),
    )(unnormalized_widths, unnormalized_heights, unnormalized_derivatives, xt)
    return out_t.T, lad_t.T
